# baseline (speedup 1.0000x reference)
import functools

import jax
import jax.numpy as jnp
from jax import lax
from jax.experimental import pallas as pl
from jax.experimental.pallas import tpu as pltpu
from jax.experimental.pallas import tpu_sc as plsc

BLK = 256
HI = jax.lax.Precision.HIGHEST


def _router_body(x_ref, rw_ref, rb_ref,
                 pos1_ref, pos2_ref, cw1_ref, cw2_ref, be_ref, bv_ref):
    N = x_ref.shape[0]
    E = rw_ref.shape[0]
    xb = x_ref[...].astype(jnp.bfloat16)
    logits = jax.lax.dot_general(
        xb, rw_ref[...], (((1,), (1,)), ((), ())),
        preferred_element_type=jnp.float32)
    lb = logits + rb_ref[...]
    ex = jnp.exp(logits - jnp.max(logits, axis=-1, keepdims=True))
    scores = ex / jnp.sum(ex, axis=-1, keepdims=True)
    i1 = jnp.argmax(lb, axis=-1, keepdims=True)
    eiota = jax.lax.broadcasted_iota(jnp.int32, logits.shape, 1)
    masked = jnp.where(eiota == i1, -jnp.inf, lb)
    i2 = jnp.argmax(masked, axis=-1, keepdims=True)
    m1 = eiota == i1
    m2 = eiota == i2
    s1 = jnp.sum(jnp.where(m1, scores, 0.0), axis=-1, keepdims=True)
    s2 = jnp.sum(jnp.where(m2, scores, 0.0), axis=-1, keepdims=True)
    denom = s1 + s2
    cw1_ref[...] = s1 / denom
    cw2_ref[...] = s2 / denom

    oh1 = m1.astype(jnp.float32)
    oh2 = m2.astype(jnp.float32)

    c1tot = jnp.sum(oh1, axis=0, keepdims=True)
    counts = c1tot + jnp.sum(oh2, axis=0, keepdims=True)
    pcount = jnp.ceil(counts / BLK) * BLK
    re = jax.lax.broadcasted_iota(jnp.int32, (E, E), 0)
    ce = jax.lax.broadcasted_iota(jnp.int32, (E, E), 1)
    u8 = (re < ce).astype(jnp.float32)
    pstart = jnp.dot(pcount, u8, precision=HI)

    r = jax.lax.broadcasted_iota(jnp.int32, (128, 128), 0)
    c = jax.lax.broadcasted_iota(jnp.int32, (128, 128), 1)
    lstrict = (r > c).astype(jnp.float32)
    off = jnp.zeros((1, E), jnp.float32)
    for i in range(N // 128):
        ch = oh1[i * 128:(i + 1) * 128]
        pref = jnp.dot(lstrict, ch, precision=HI) + off + pstart
        pos = jnp.sum(ch * pref, axis=-1, keepdims=True)
        pos1_ref[pl.ds(i * 128, 128), :] = pos.astype(jnp.int32)
        off = off + jnp.sum(ch, axis=0, keepdims=True)
    for i in range(N // 128):
        ch = oh2[i * 128:(i + 1) * 128]
        pref = jnp.dot(lstrict, ch, precision=HI) + off + pstart
        pos = jnp.sum(ch * pref, axis=-1, keepdims=True)
        pos2_ref[pl.ds(i * 128, 128), :] = pos.astype(jnp.int32)
        off = off + jnp.sum(ch, axis=0, keepdims=True)

    pend = pstart + pcount
    biota = jax.lax.broadcasted_iota(
        jnp.int32, (1, 128), 1).astype(jnp.float32) * BLK
    be = jnp.zeros((1, 128), jnp.float32)
    for e in range(E):
        be = be + (biota >= pend[0, e]).astype(jnp.float32)
    be_ref[...] = jnp.minimum(be, E - 1).astype(jnp.int32)
    bv_ref[...] = (biota < pend[0, E - 1]).astype(jnp.int32)


def _shared_body(x_ref, sg_ref, su_ref, sd_ref, r1_ref, r2_ref, out_ref):
    xb = x_ref[...].astype(jnp.bfloat16)
    gs = jax.lax.dot_general(xb, sg_ref[...], (((1,), (1,)), ((), ())),
                             preferred_element_type=jnp.float32)
    us = jax.lax.dot_general(xb, su_ref[...], (((1,), (1,)), ((), ())),
                             preferred_element_type=jnp.float32)
    hs = (gs * jax.lax.logistic(gs) * us).astype(jnp.bfloat16)
    out_ref[...] = r1_ref[...] + r2_ref[...] + jax.lax.dot_general(
        hs, sd_ref[...], (((1,), (0,)), ((), ())),
        preferred_element_type=jnp.float32)


def _grouped_body(be_ref, bv_ref, xs_ref, cws_ref, g_ref, u_ref, d_ref,
                  out_ref):
    b = pl.program_id(0)

    @pl.when(bv_ref[b] == 1)
    def _():
        xb = xs_ref[...].astype(jnp.bfloat16)
        g = jax.lax.dot_general(xb, g_ref[0], (((1,), (1,)), ((), ())),
                                preferred_element_type=jnp.float32)
        u = jax.lax.dot_general(xb, u_ref[0], (((1,), (1,)), ((), ())),
                                preferred_element_type=jnp.float32)
        h = (g * jax.lax.logistic(g) * u).astype(jnp.bfloat16)
        eo = jax.lax.dot_general(h, d_ref[0], (((1,), (0,)), ((), ())),
                                 preferred_element_type=jnp.float32)
        out_ref[...] = eo * cws_ref[...]


def _sc_scatter_body(x_hbm, pos1_hbm, pos2_hbm, cw1_hbm, cw2_hbm,
                     xs_hbm, cws_hbm,
                     idx1_v, idx2_v, w_v, rows_v, sem):
    nc = 2
    wid = lax.axis_index("s") * nc + lax.axis_index("c")
    for half in range(2):
        base = wid * 128 + half * 64
        pltpu.sync_copy(pos1_hbm.at[pl.ds(base, 64)], idx1_v)
        pltpu.sync_copy(pos2_hbm.at[pl.ds(base, 64)], idx2_v)
        pltpu.sync_copy(x_hbm.at[pl.ds(base, 64)], rows_v)
        pltpu.async_copy(rows_v, xs_hbm.at[idx1_v], sem).wait()
        pltpu.async_copy(rows_v, xs_hbm.at[idx2_v], sem).wait()
        pltpu.sync_copy(cw1_hbm.at[pl.ds(base, 64)], w_v)
        pltpu.async_copy(w_v, cws_hbm.at[idx1_v], sem).wait()
        pltpu.sync_copy(cw2_hbm.at[pl.ds(base, 64)], w_v)
        pltpu.async_copy(w_v, cws_hbm.at[idx2_v], sem).wait()


def _sc_gather_body(eo_hbm, pos1_hbm, pos2_hbm, r1_hbm, r2_hbm,
                    idx_v, rows_v, sem):
    nc = 2
    wid = lax.axis_index("s") * nc + lax.axis_index("c")
    for half in range(2):
        base = wid * 128 + half * 64
        pltpu.sync_copy(pos1_hbm.at[pl.ds(base, 64)], idx_v)
        pltpu.async_copy(eo_hbm.at[idx_v], rows_v, sem).wait()
        pltpu.sync_copy(rows_v, r1_hbm.at[pl.ds(base, 64)])
        pltpu.sync_copy(pos2_hbm.at[pl.ds(base, 64)], idx_v)
        pltpu.async_copy(eo_hbm.at[idx_v], rows_v, sem).wait()
        pltpu.sync_copy(rows_v, r2_hbm.at[pl.ds(base, 64)])


@jax.jit
def kernel(x, router_w, router_bias, gate_w, up_w, down_w,
           shared_gate_w, shared_up_w, shared_down_w):
    Bs, Ts, D = x.shape
    N = Bs * Ts
    E, H, _ = gate_w.shape
    NS, SH, _ = shared_gate_w.shape
    NB = N * 2 // BLK + E - 1
    NB = ((NB + 7) // 8) * 8
    P = NB * BLK
    flat = x.reshape(N, D)

    bf = jnp.bfloat16
    rw = router_w.astype(bf)
    gw = gate_w.astype(bf)
    uw = up_w.astype(bf)
    dw = jnp.swapaxes(down_w, 1, 2).astype(bf)
    sg = shared_gate_w.reshape(NS * SH, D).astype(bf)
    su = shared_up_w.reshape(NS * SH, D).astype(bf)
    sd = jnp.swapaxes(shared_down_w, 1, 2).reshape(NS * SH, D).astype(bf)

    pos1, pos2, cw1, cw2, be2, bv2 = pl.pallas_call(
        _router_body,
        grid=(1,),
        in_specs=[
            pl.BlockSpec((N, D), lambda i: (0, 0)),
            pl.BlockSpec((E, D), lambda i: (0, 0)),
            pl.BlockSpec((E,), lambda i: (0,)),
        ],
        out_specs=[
            pl.BlockSpec((N, 1), lambda i: (0, 0)),
            pl.BlockSpec((N, 1), lambda i: (0, 0)),
            pl.BlockSpec((N, 1), lambda i: (0, 0)),
            pl.BlockSpec((N, 1), lambda i: (0, 0)),
            pl.BlockSpec((1, 128), lambda i: (0, 0)),
            pl.BlockSpec((1, 128), lambda i: (0, 0)),
        ],
        out_shape=[
            jax.ShapeDtypeStruct((N, 1), jnp.int32),
            jax.ShapeDtypeStruct((N, 1), jnp.int32),
            jax.ShapeDtypeStruct((N, 1), jnp.float32),
            jax.ShapeDtypeStruct((N, 1), jnp.float32),
            jax.ShapeDtypeStruct((1, 128), jnp.int32),
            jax.ShapeDtypeStruct((1, 128), jnp.int32),
        ],
    )(flat, rw, router_bias)
    pos1f = pos1.reshape(N)
    pos2f = pos2.reshape(N)
    be = be2.reshape(128)[:NB]
    bv = bv2.reshape(128)[:NB]

    mesh = plsc.VectorSubcoreMesh(core_axis_name="c", subcore_axis_name="s")
    xs, cws = pl.kernel(
        _sc_scatter_body,
        out_type=(
            jax.ShapeDtypeStruct((P, D), jnp.float32),
            jax.ShapeDtypeStruct((P,), jnp.float32),
        ),
        mesh=mesh,
        scratch_types=[
            pltpu.VMEM((64,), jnp.int32),
            pltpu.VMEM((64,), jnp.int32),
            pltpu.VMEM((64,), jnp.float32),
            pltpu.VMEM((64, D), jnp.float32),
            pltpu.SemaphoreType.DMA,
        ],
    )(flat, pos1f, pos2f, cw1.reshape(N), cw2.reshape(N))

    eo2 = pl.pallas_call(
        _grouped_body,
        grid_spec=pltpu.PrefetchScalarGridSpec(
            num_scalar_prefetch=2,
            grid=(NB,),
            in_specs=[
                pl.BlockSpec((BLK, D), lambda b, be_r, bv_r: (b, 0)),
                pl.BlockSpec((BLK, 1), lambda b, be_r, bv_r: (b, 0)),
                pl.BlockSpec((1, H, D), lambda b, be_r, bv_r: (be_r[b], 0, 0)),
                pl.BlockSpec((1, H, D), lambda b, be_r, bv_r: (be_r[b], 0, 0)),
                pl.BlockSpec((1, H, D), lambda b, be_r, bv_r: (be_r[b], 0, 0)),
            ],
            out_specs=pl.BlockSpec((BLK, D), lambda b, be_r, bv_r: (b, 0)),
        ),
        out_shape=jax.ShapeDtypeStruct((P, D), jnp.float32),
    )(be, bv, xs, cws.reshape(P, 1), gw, uw, dw)

    r1, r2 = pl.kernel(
        _sc_gather_body,
        out_type=(
            jax.ShapeDtypeStruct((N, D), jnp.float32),
            jax.ShapeDtypeStruct((N, D), jnp.float32),
        ),
        mesh=mesh,
        scratch_types=[
            pltpu.VMEM((64,), jnp.int32),
            pltpu.VMEM((64, D), jnp.float32),
            pltpu.SemaphoreType.DMA,
        ],
    )(eo2, pos1f, pos2f)

    TB = 512
    out = pl.pallas_call(
        _shared_body,
        grid=(N // TB,),
        in_specs=[
            pl.BlockSpec((TB, D), lambda i: (i, 0)),
            pl.BlockSpec((NS * SH, D), lambda i: (0, 0)),
            pl.BlockSpec((NS * SH, D), lambda i: (0, 0)),
            pl.BlockSpec((NS * SH, D), lambda i: (0, 0)),
            pl.BlockSpec((TB, D), lambda i: (i, 0)),
            pl.BlockSpec((TB, D), lambda i: (i, 0)),
        ],
        out_specs=pl.BlockSpec((TB, D), lambda i: (i, 0)),
        out_shape=jax.ShapeDtypeStruct((N, D), jnp.float32),
    )(flat, sg, su, sd, r1, r2)

    return out.reshape(Bs, Ts, D)


# speedup vs baseline: 1.1307x; 1.1307x over previous
import functools

import jax
import jax.numpy as jnp
from jax import lax
from jax.experimental import pallas as pl
from jax.experimental.pallas import tpu as pltpu
from jax.experimental.pallas import tpu_sc as plsc

BLK = 256
HI = jax.lax.Precision.HIGHEST


def _router_body(x_ref, rw_ref, rb_ref,
                 pos1_ref, pos2_ref, cw1_ref, cw2_ref, be_ref, bv_ref):
    N = x_ref.shape[0]
    E = rw_ref.shape[0]
    xb = x_ref[...].astype(jnp.bfloat16)
    logits = jax.lax.dot_general(
        xb, rw_ref[...], (((1,), (1,)), ((), ())),
        preferred_element_type=jnp.float32)
    lb = logits + rb_ref[...]
    ex = jnp.exp(logits - jnp.max(logits, axis=-1, keepdims=True))
    scores = ex / jnp.sum(ex, axis=-1, keepdims=True)
    i1 = jnp.argmax(lb, axis=-1, keepdims=True)
    eiota = jax.lax.broadcasted_iota(jnp.int32, logits.shape, 1)
    masked = jnp.where(eiota == i1, -jnp.inf, lb)
    i2 = jnp.argmax(masked, axis=-1, keepdims=True)
    m1 = eiota == i1
    m2 = eiota == i2
    s1 = jnp.sum(jnp.where(m1, scores, 0.0), axis=-1, keepdims=True)
    s2 = jnp.sum(jnp.where(m2, scores, 0.0), axis=-1, keepdims=True)
    denom = s1 + s2
    cw1_ref[...] = s1 / denom
    cw2_ref[...] = s2 / denom

    oh1 = m1.astype(jnp.float32)
    oh2 = m2.astype(jnp.float32)

    c1tot = jnp.sum(oh1, axis=0, keepdims=True)
    counts = c1tot + jnp.sum(oh2, axis=0, keepdims=True)
    pcount = jnp.ceil(counts / BLK) * BLK
    re = jax.lax.broadcasted_iota(jnp.int32, (E, E), 0)
    ce = jax.lax.broadcasted_iota(jnp.int32, (E, E), 1)
    u8 = (re < ce).astype(jnp.float32)
    pstart = jnp.dot(pcount, u8, precision=HI)

    r = jax.lax.broadcasted_iota(jnp.int32, (128, 128), 0)
    c = jax.lax.broadcasted_iota(jnp.int32, (128, 128), 1)
    lstrict = (r > c).astype(jnp.float32)
    off = jnp.zeros((1, E), jnp.float32)
    for i in range(N // 128):
        ch = oh1[i * 128:(i + 1) * 128]
        pref = jnp.dot(lstrict, ch, precision=HI) + off + pstart
        pos = jnp.sum(ch * pref, axis=-1, keepdims=True)
        pos1_ref[pl.ds(i * 128, 128), :] = pos.astype(jnp.int32)
        off = off + jnp.sum(ch, axis=0, keepdims=True)
    for i in range(N // 128):
        ch = oh2[i * 128:(i + 1) * 128]
        pref = jnp.dot(lstrict, ch, precision=HI) + off + pstart
        pos = jnp.sum(ch * pref, axis=-1, keepdims=True)
        pos2_ref[pl.ds(i * 128, 128), :] = pos.astype(jnp.int32)
        off = off + jnp.sum(ch, axis=0, keepdims=True)

    pend = pstart + pcount
    biota = jax.lax.broadcasted_iota(
        jnp.int32, (1, 128), 1).astype(jnp.float32) * BLK
    be = jnp.zeros((1, 128), jnp.float32)
    for e in range(E):
        be = be + (biota >= pend[0, e]).astype(jnp.float32)
    be_ref[...] = jnp.minimum(be, E - 1).astype(jnp.int32)
    bv_ref[...] = (biota < pend[0, E - 1]).astype(jnp.int32)


def _shared_body(x_ref, sg_ref, su_ref, sd_ref, r1_ref, r2_ref,
                 cw1_ref, cw2_ref, out_ref):
    xb = x_ref[...].astype(jnp.bfloat16)
    gs = jax.lax.dot_general(xb, sg_ref[...], (((1,), (1,)), ((), ())),
                             preferred_element_type=jnp.float32)
    us = jax.lax.dot_general(xb, su_ref[...], (((1,), (1,)), ((), ())),
                             preferred_element_type=jnp.float32)
    hs = (gs * jax.lax.logistic(gs) * us).astype(jnp.bfloat16)
    out_ref[...] = (cw1_ref[...] * r1_ref[...] + cw2_ref[...] * r2_ref[...]
                    + jax.lax.dot_general(
                        hs, sd_ref[...], (((1,), (0,)), ((), ())),
                        preferred_element_type=jnp.float32))


def _grouped_body(be_ref, bv_ref, xs_ref, g_ref, u_ref, d_ref, out_ref):
    b = pl.program_id(0)

    @pl.when(bv_ref[b] == 1)
    def _():
        xb = xs_ref[...].astype(jnp.bfloat16)
        g = jax.lax.dot_general(xb, g_ref[0], (((1,), (1,)), ((), ())),
                                preferred_element_type=jnp.float32)
        u = jax.lax.dot_general(xb, u_ref[0], (((1,), (1,)), ((), ())),
                                preferred_element_type=jnp.float32)
        h = (g * jax.lax.logistic(g) * u).astype(jnp.bfloat16)
        out_ref[...] = jax.lax.dot_general(
            h, d_ref[0], (((1,), (0,)), ((), ())),
            preferred_element_type=jnp.float32)


def _sc_scatter_body(x_hbm, pos1_hbm, pos2_hbm, xs_hbm,
                     idx1a_v, idx1b_v, idx2a_v, idx2b_v,
                     rowsa_v, rowsb_v, sema, semb):
    nc = 2
    wid = lax.axis_index("s") * nc + lax.axis_index("c")
    idx1 = (idx1a_v, idx1b_v)
    idx2 = (idx2a_v, idx2b_v)
    rows = (rowsa_v, rowsb_v)
    sems = (sema, semb)
    pend = [None] * 4
    for c in range(4):
        sl = c % 2
        if c >= 2:
            pend[c - 2][0].wait()
            pend[c - 2][1].wait()
        base = wid * 128 + c * 32
        pltpu.sync_copy(pos1_hbm.at[pl.ds(base, 32)], idx1[sl])
        pltpu.sync_copy(pos2_hbm.at[pl.ds(base, 32)], idx2[sl])
        pltpu.sync_copy(x_hbm.at[pl.ds(base, 32)], rows[sl])
        pend[c] = (
            pltpu.async_copy(rows[sl], xs_hbm.at[idx1[sl]], sems[sl]),
            pltpu.async_copy(rows[sl], xs_hbm.at[idx2[sl]], sems[sl]),
        )
    for c in (2, 3):
        pend[c][0].wait()
        pend[c][1].wait()


def _sc_gather_body(eo_hbm, pos1_hbm, pos2_hbm, r1_hbm, r2_hbm,
                    idxa_v, idxb_v, rowsa_v, rowsb_v,
                    semga, semgb, semwa, semwb):
    nc = 2
    wid = lax.axis_index("s") * nc + lax.axis_index("c")
    idxs = (idxa_v, idxb_v)
    rows = (rowsa_v, rowsb_v)
    semg = (semga, semgb)
    semw = (semwa, semwb)
    units = [(ch, k) for ch in range(4) for k in range(2)]
    gd = [None] * 8
    wd = [None] * 8
    for u, (ch, k) in enumerate(units):
        sl = u % 2
        if u >= 2:
            wd[u - 2].wait()
        base = wid * 128 + ch * 32
        pos_hbm = pos1_hbm if k == 0 else pos2_hbm
        pltpu.sync_copy(pos_hbm.at[pl.ds(base, 32)], idxs[sl])
        gd[u] = pltpu.async_copy(eo_hbm.at[idxs[sl]], rows[sl], semg[sl])
        if u >= 1:
            pu = u - 1
            psl = pu % 2
            pch, pk = units[pu]
            pbase = wid * 128 + pch * 32
            pr_hbm = r1_hbm if pk == 0 else r2_hbm
            gd[pu].wait()
            wd[pu] = pltpu.async_copy(rows[psl], pr_hbm.at[pl.ds(pbase, 32)],
                                      semw[psl])
    gd[7].wait()
    wd[7] = pltpu.async_copy(rows[7 % 2], r2_hbm.at[pl.ds(wid * 128 + 96, 32)],
                             semw[7 % 2])
    wd[6].wait()
    wd[7].wait()


@jax.jit
def kernel(x, router_w, router_bias, gate_w, up_w, down_w,
           shared_gate_w, shared_up_w, shared_down_w):
    Bs, Ts, D = x.shape
    N = Bs * Ts
    E, H, _ = gate_w.shape
    NS, SH, _ = shared_gate_w.shape
    NB = N * 2 // BLK + E - 1
    NB = ((NB + 7) // 8) * 8
    P = NB * BLK
    flat = x.reshape(N, D)

    bf = jnp.bfloat16
    rw = router_w.astype(bf)
    gw = gate_w.astype(bf)
    uw = up_w.astype(bf)
    dw = jnp.swapaxes(down_w, 1, 2).astype(bf)
    sg = shared_gate_w.reshape(NS * SH, D).astype(bf)
    su = shared_up_w.reshape(NS * SH, D).astype(bf)
    sd = jnp.swapaxes(shared_down_w, 1, 2).reshape(NS * SH, D).astype(bf)

    pos1, pos2, cw1, cw2, be2, bv2 = pl.pallas_call(
        _router_body,
        grid=(1,),
        in_specs=[
            pl.BlockSpec((N, D), lambda i: (0, 0)),
            pl.BlockSpec((E, D), lambda i: (0, 0)),
            pl.BlockSpec((E,), lambda i: (0,)),
        ],
        out_specs=[
            pl.BlockSpec((N, 1), lambda i: (0, 0)),
            pl.BlockSpec((N, 1), lambda i: (0, 0)),
            pl.BlockSpec((N, 1), lambda i: (0, 0)),
            pl.BlockSpec((N, 1), lambda i: (0, 0)),
            pl.BlockSpec((1, 128), lambda i: (0, 0)),
            pl.BlockSpec((1, 128), lambda i: (0, 0)),
        ],
        out_shape=[
            jax.ShapeDtypeStruct((N, 1), jnp.int32),
            jax.ShapeDtypeStruct((N, 1), jnp.int32),
            jax.ShapeDtypeStruct((N, 1), jnp.float32),
            jax.ShapeDtypeStruct((N, 1), jnp.float32),
            jax.ShapeDtypeStruct((1, 128), jnp.int32),
            jax.ShapeDtypeStruct((1, 128), jnp.int32),
        ],
    )(flat, rw, router_bias)
    pos1f = pos1.reshape(N)
    pos2f = pos2.reshape(N)
    be = be2.reshape(128)[:NB]
    bv = bv2.reshape(128)[:NB]

    mesh = plsc.VectorSubcoreMesh(core_axis_name="c", subcore_axis_name="s")
    xs = pl.kernel(
        _sc_scatter_body,
        out_type=jax.ShapeDtypeStruct((P, D), jnp.float32),
        mesh=mesh,
        scratch_types=[
            pltpu.VMEM((32,), jnp.int32),
            pltpu.VMEM((32,), jnp.int32),
            pltpu.VMEM((32,), jnp.int32),
            pltpu.VMEM((32,), jnp.int32),
            pltpu.VMEM((32, D), jnp.float32),
            pltpu.VMEM((32, D), jnp.float32),
            pltpu.SemaphoreType.DMA,
            pltpu.SemaphoreType.DMA,
        ],
    )(flat, pos1f, pos2f)

    eo2 = pl.pallas_call(
        _grouped_body,
        grid_spec=pltpu.PrefetchScalarGridSpec(
            num_scalar_prefetch=2,
            grid=(NB,),
            in_specs=[
                pl.BlockSpec((BLK, D), lambda b, be_r, bv_r: (b, 0)),
                pl.BlockSpec((1, H, D), lambda b, be_r, bv_r: (be_r[b], 0, 0)),
                pl.BlockSpec((1, H, D), lambda b, be_r, bv_r: (be_r[b], 0, 0)),
                pl.BlockSpec((1, H, D), lambda b, be_r, bv_r: (be_r[b], 0, 0)),
            ],
            out_specs=pl.BlockSpec((BLK, D), lambda b, be_r, bv_r: (b, 0)),
        ),
        out_shape=jax.ShapeDtypeStruct((P, D), jnp.float32),
    )(be, bv, xs, gw, uw, dw)

    r1, r2 = pl.kernel(
        _sc_gather_body,
        out_type=(
            jax.ShapeDtypeStruct((N, D), jnp.float32),
            jax.ShapeDtypeStruct((N, D), jnp.float32),
        ),
        mesh=mesh,
        scratch_types=[
            pltpu.VMEM((32,), jnp.int32),
            pltpu.VMEM((32,), jnp.int32),
            pltpu.VMEM((32, D), jnp.float32),
            pltpu.VMEM((32, D), jnp.float32),
            pltpu.SemaphoreType.DMA,
            pltpu.SemaphoreType.DMA,
            pltpu.SemaphoreType.DMA,
            pltpu.SemaphoreType.DMA,
        ],
    )(eo2, pos1f, pos2f)

    TB = 512
    out = pl.pallas_call(
        _shared_body,
        grid=(N // TB,),
        in_specs=[
            pl.BlockSpec((TB, D), lambda i: (i, 0)),
            pl.BlockSpec((NS * SH, D), lambda i: (0, 0)),
            pl.BlockSpec((NS * SH, D), lambda i: (0, 0)),
            pl.BlockSpec((NS * SH, D), lambda i: (0, 0)),
            pl.BlockSpec((TB, D), lambda i: (i, 0)),
            pl.BlockSpec((TB, D), lambda i: (i, 0)),
            pl.BlockSpec((TB, 1), lambda i: (i, 0)),
            pl.BlockSpec((TB, 1), lambda i: (i, 0)),
        ],
        out_specs=pl.BlockSpec((TB, D), lambda i: (i, 0)),
        out_shape=jax.ShapeDtypeStruct((N, D), jnp.float32),
    )(flat, sg, su, sd, r1, r2, cw1, cw2)

    return out.reshape(Bs, Ts, D)


# speedup vs baseline: 1.1429x; 1.0108x over previous
import functools

import jax
import jax.numpy as jnp
from jax import lax
from jax.experimental import pallas as pl
from jax.experimental.pallas import tpu as pltpu
from jax.experimental.pallas import tpu_sc as plsc

BLK = 256
HI = jax.lax.Precision.HIGHEST


def _router_body(x_ref, rw_ref, rb_ref,
                 pos1_ref, pos2_ref, cw1_ref, cw2_ref, be_ref, bv_ref):
    N = x_ref.shape[0]
    E = rw_ref.shape[0]
    xb = x_ref[...].astype(jnp.bfloat16)
    logits = jax.lax.dot_general(
        xb, rw_ref[...], (((1,), (1,)), ((), ())),
        preferred_element_type=jnp.float32)
    lb = logits + rb_ref[...]
    ex = jnp.exp(logits - jnp.max(logits, axis=-1, keepdims=True))
    scores = ex / jnp.sum(ex, axis=-1, keepdims=True)
    i1 = jnp.argmax(lb, axis=-1, keepdims=True)
    eiota = jax.lax.broadcasted_iota(jnp.int32, logits.shape, 1)
    masked = jnp.where(eiota == i1, -jnp.inf, lb)
    i2 = jnp.argmax(masked, axis=-1, keepdims=True)
    m1 = eiota == i1
    m2 = eiota == i2
    s1 = jnp.sum(jnp.where(m1, scores, 0.0), axis=-1, keepdims=True)
    s2 = jnp.sum(jnp.where(m2, scores, 0.0), axis=-1, keepdims=True)
    denom = s1 + s2
    cw1_ref[...] = s1 / denom
    cw2_ref[...] = s2 / denom

    oh1 = m1.astype(jnp.float32)
    oh2 = m2.astype(jnp.float32)

    c1tot = jnp.sum(oh1, axis=0, keepdims=True)
    counts = c1tot + jnp.sum(oh2, axis=0, keepdims=True)
    pcount = jnp.ceil(counts / BLK) * BLK
    re = jax.lax.broadcasted_iota(jnp.int32, (E, E), 0)
    ce = jax.lax.broadcasted_iota(jnp.int32, (E, E), 1)
    u8 = (re < ce).astype(jnp.float32)
    pstart = jnp.dot(pcount, u8, precision=HI)

    CH = 256
    r = jax.lax.broadcasted_iota(jnp.int32, (CH, CH), 0)
    c = jax.lax.broadcasted_iota(jnp.int32, (CH, CH), 1)
    lstrict = (r > c).astype(jnp.bfloat16)
    off1 = jnp.zeros((1, E), jnp.float32)
    off2 = c1tot
    for i in range(N // CH):
        ch1 = oh1[i * CH:(i + 1) * CH]
        ch2 = oh2[i * CH:(i + 1) * CH]
        p1 = jnp.dot(lstrict, ch1.astype(jnp.bfloat16),
                     preferred_element_type=jnp.float32) + (off1 + pstart)
        p2 = jnp.dot(lstrict, ch2.astype(jnp.bfloat16),
                     preferred_element_type=jnp.float32) + (off2 + pstart)
        pos1_ref[pl.ds(i * CH, CH), :] = jnp.sum(
            ch1 * p1, axis=-1, keepdims=True).astype(jnp.int32)
        pos2_ref[pl.ds(i * CH, CH), :] = jnp.sum(
            ch2 * p2, axis=-1, keepdims=True).astype(jnp.int32)
        off1 = off1 + jnp.sum(ch1, axis=0, keepdims=True)
        off2 = off2 + jnp.sum(ch2, axis=0, keepdims=True)

    pend = pstart + pcount
    biota = jax.lax.broadcasted_iota(
        jnp.int32, (1, 128), 1).astype(jnp.float32) * BLK
    be = jnp.zeros((1, 128), jnp.float32)
    for e in range(E):
        be = be + (biota >= pend[0, e]).astype(jnp.float32)
    be_ref[...] = jnp.minimum(be, E - 1).astype(jnp.int32)
    bv_ref[...] = (biota < pend[0, E - 1]).astype(jnp.int32)


def _shared_body(x_ref, sg_ref, su_ref, sd_ref, r1_ref, r2_ref,
                 cw1_ref, cw2_ref, out_ref):
    xb = x_ref[...].astype(jnp.bfloat16)
    gs = jax.lax.dot_general(xb, sg_ref[...], (((1,), (1,)), ((), ())),
                             preferred_element_type=jnp.float32)
    us = jax.lax.dot_general(xb, su_ref[...], (((1,), (1,)), ((), ())),
                             preferred_element_type=jnp.float32)
    hs = (gs * jax.lax.logistic(gs) * us).astype(jnp.bfloat16)
    out_ref[...] = (cw1_ref[...] * r1_ref[...] + cw2_ref[...] * r2_ref[...]
                    + jax.lax.dot_general(
                        hs, sd_ref[...], (((1,), (0,)), ((), ())),
                        preferred_element_type=jnp.float32))


def _grouped_body(be_ref, bv_ref, xs_ref, g_ref, u_ref, d_ref, out_ref):
    b = pl.program_id(0)

    @pl.when(bv_ref[b] == 1)
    def _():
        xb = xs_ref[...].astype(jnp.bfloat16)
        g = jax.lax.dot_general(xb, g_ref[0], (((1,), (1,)), ((), ())),
                                preferred_element_type=jnp.float32)
        u = jax.lax.dot_general(xb, u_ref[0], (((1,), (1,)), ((), ())),
                                preferred_element_type=jnp.float32)
        h = (g * jax.lax.logistic(g) * u).astype(jnp.bfloat16)
        out_ref[...] = jax.lax.dot_general(
            h, d_ref[0], (((1,), (0,)), ((), ())),
            preferred_element_type=jnp.float32)


def _sc_scatter_body(x_hbm, pos1_hbm, pos2_hbm, xs_hbm,
                     idx1a_v, idx1b_v, idx2a_v, idx2b_v,
                     rowsa_v, rowsb_v, sema, semb):
    nc = 2
    wid = lax.axis_index("s") * nc + lax.axis_index("c")
    idx1 = (idx1a_v, idx1b_v)
    idx2 = (idx2a_v, idx2b_v)
    rows = (rowsa_v, rowsb_v)
    sems = (sema, semb)
    pend = [None] * 4
    for c in range(4):
        sl = c % 2
        if c >= 2:
            pend[c - 2][0].wait()
            pend[c - 2][1].wait()
        base = wid * 128 + c * 32
        pltpu.sync_copy(pos1_hbm.at[pl.ds(base, 32)], idx1[sl])
        pltpu.sync_copy(pos2_hbm.at[pl.ds(base, 32)], idx2[sl])
        pltpu.sync_copy(x_hbm.at[pl.ds(base, 32)], rows[sl])
        pend[c] = (
            pltpu.async_copy(rows[sl], xs_hbm.at[idx1[sl]], sems[sl]),
            pltpu.async_copy(rows[sl], xs_hbm.at[idx2[sl]], sems[sl]),
        )
    for c in (2, 3):
        pend[c][0].wait()
        pend[c][1].wait()


def _sc_gather_body(eo_hbm, pos1_hbm, pos2_hbm, r1_hbm, r2_hbm,
                    idxa_v, idxb_v, rowsa_v, rowsb_v,
                    semga, semgb, semwa, semwb):
    nc = 2
    wid = lax.axis_index("s") * nc + lax.axis_index("c")
    idxs = (idxa_v, idxb_v)
    rows = (rowsa_v, rowsb_v)
    semg = (semga, semgb)
    semw = (semwa, semwb)
    units = [(ch, k) for ch in range(4) for k in range(2)]
    gd = [None] * 8
    wd = [None] * 8
    for u, (ch, k) in enumerate(units):
        sl = u % 2
        if u >= 2:
            wd[u - 2].wait()
        base = wid * 128 + ch * 32
        pos_hbm = pos1_hbm if k == 0 else pos2_hbm
        pltpu.sync_copy(pos_hbm.at[pl.ds(base, 32)], idxs[sl])
        gd[u] = pltpu.async_copy(eo_hbm.at[idxs[sl]], rows[sl], semg[sl])
        if u >= 1:
            pu = u - 1
            psl = pu % 2
            pch, pk = units[pu]
            pbase = wid * 128 + pch * 32
            pr_hbm = r1_hbm if pk == 0 else r2_hbm
            gd[pu].wait()
            wd[pu] = pltpu.async_copy(rows[psl], pr_hbm.at[pl.ds(pbase, 32)],
                                      semw[psl])
    gd[7].wait()
    wd[7] = pltpu.async_copy(rows[7 % 2], r2_hbm.at[pl.ds(wid * 128 + 96, 32)],
                             semw[7 % 2])
    wd[6].wait()
    wd[7].wait()


@jax.jit
def kernel(x, router_w, router_bias, gate_w, up_w, down_w,
           shared_gate_w, shared_up_w, shared_down_w):
    Bs, Ts, D = x.shape
    N = Bs * Ts
    E, H, _ = gate_w.shape
    NS, SH, _ = shared_gate_w.shape
    NB = N * 2 // BLK + E - 1
    NB = ((NB + 7) // 8) * 8
    P = NB * BLK
    flat = x.reshape(N, D)

    bf = jnp.bfloat16
    rw = router_w.astype(bf)
    gw = gate_w.astype(bf)
    uw = up_w.astype(bf)
    dw = jnp.swapaxes(down_w, 1, 2).astype(bf)
    sg = shared_gate_w.reshape(NS * SH, D).astype(bf)
    su = shared_up_w.reshape(NS * SH, D).astype(bf)
    sd = jnp.swapaxes(shared_down_w, 1, 2).reshape(NS * SH, D).astype(bf)

    pos1, pos2, cw1, cw2, be2, bv2 = pl.pallas_call(
        _router_body,
        grid=(1,),
        in_specs=[
            pl.BlockSpec((N, D), lambda i: (0, 0)),
            pl.BlockSpec((E, D), lambda i: (0, 0)),
            pl.BlockSpec((E,), lambda i: (0,)),
        ],
        out_specs=[
            pl.BlockSpec((N, 1), lambda i: (0, 0)),
            pl.BlockSpec((N, 1), lambda i: (0, 0)),
            pl.BlockSpec((N, 1), lambda i: (0, 0)),
            pl.BlockSpec((N, 1), lambda i: (0, 0)),
            pl.BlockSpec((1, 128), lambda i: (0, 0)),
            pl.BlockSpec((1, 128), lambda i: (0, 0)),
        ],
        out_shape=[
            jax.ShapeDtypeStruct((N, 1), jnp.int32),
            jax.ShapeDtypeStruct((N, 1), jnp.int32),
            jax.ShapeDtypeStruct((N, 1), jnp.float32),
            jax.ShapeDtypeStruct((N, 1), jnp.float32),
            jax.ShapeDtypeStruct((1, 128), jnp.int32),
            jax.ShapeDtypeStruct((1, 128), jnp.int32),
        ],
    )(flat, rw, router_bias)
    pos1f = pos1.reshape(N)
    pos2f = pos2.reshape(N)
    be = be2.reshape(128)[:NB]
    bv = bv2.reshape(128)[:NB]

    mesh = plsc.VectorSubcoreMesh(core_axis_name="c", subcore_axis_name="s")
    xs = pl.kernel(
        _sc_scatter_body,
        out_type=jax.ShapeDtypeStruct((P, D), jnp.float32),
        mesh=mesh,
        scratch_types=[
            pltpu.VMEM((32,), jnp.int32),
            pltpu.VMEM((32,), jnp.int32),
            pltpu.VMEM((32,), jnp.int32),
            pltpu.VMEM((32,), jnp.int32),
            pltpu.VMEM((32, D), jnp.float32),
            pltpu.VMEM((32, D), jnp.float32),
            pltpu.SemaphoreType.DMA,
            pltpu.SemaphoreType.DMA,
        ],
    )(flat, pos1f, pos2f)

    eo2 = pl.pallas_call(
        _grouped_body,
        grid_spec=pltpu.PrefetchScalarGridSpec(
            num_scalar_prefetch=2,
            grid=(NB,),
            in_specs=[
                pl.BlockSpec((BLK, D), lambda b, be_r, bv_r: (b, 0)),
                pl.BlockSpec((1, H, D), lambda b, be_r, bv_r: (be_r[b], 0, 0)),
                pl.BlockSpec((1, H, D), lambda b, be_r, bv_r: (be_r[b], 0, 0)),
                pl.BlockSpec((1, H, D), lambda b, be_r, bv_r: (be_r[b], 0, 0)),
            ],
            out_specs=pl.BlockSpec((BLK, D), lambda b, be_r, bv_r: (b, 0)),
        ),
        out_shape=jax.ShapeDtypeStruct((P, D), jnp.float32),
    )(be, bv, xs, gw, uw, dw)

    r1, r2 = pl.kernel(
        _sc_gather_body,
        out_type=(
            jax.ShapeDtypeStruct((N, D), jnp.float32),
            jax.ShapeDtypeStruct((N, D), jnp.float32),
        ),
        mesh=mesh,
        scratch_types=[
            pltpu.VMEM((32,), jnp.int32),
            pltpu.VMEM((32,), jnp.int32),
            pltpu.VMEM((32, D), jnp.float32),
            pltpu.VMEM((32, D), jnp.float32),
            pltpu.SemaphoreType.DMA,
            pltpu.SemaphoreType.DMA,
            pltpu.SemaphoreType.DMA,
            pltpu.SemaphoreType.DMA,
        ],
    )(eo2, pos1f, pos2f)

    TB = 512
    out = pl.pallas_call(
        _shared_body,
        grid=(N // TB,),
        in_specs=[
            pl.BlockSpec((TB, D), lambda i: (i, 0)),
            pl.BlockSpec((NS * SH, D), lambda i: (0, 0)),
            pl.BlockSpec((NS * SH, D), lambda i: (0, 0)),
            pl.BlockSpec((NS * SH, D), lambda i: (0, 0)),
            pl.BlockSpec((TB, D), lambda i: (i, 0)),
            pl.BlockSpec((TB, D), lambda i: (i, 0)),
            pl.BlockSpec((TB, 1), lambda i: (i, 0)),
            pl.BlockSpec((TB, 1), lambda i: (i, 0)),
        ],
        out_specs=pl.BlockSpec((TB, D), lambda i: (i, 0)),
        out_shape=jax.ShapeDtypeStruct((N, D), jnp.float32),
    )(flat, sg, su, sd, r1, r2, cw1, cw2)

    return out.reshape(Bs, Ts, D)
